# baseline (speedup 1.0000x reference)
import functools

import jax
import jax.numpy as jnp
from jax.experimental import pallas as pl
from jax.experimental.pallas import tpu as pltpu

_M_BLK = 128
_K_BLK = 4096


def _mm_body(x_ref, w_ref, o_ref, *, k_total, k_blk, nk):
    k = pl.program_id(1)

    @pl.when(k == 0)
    def _init():
        o_ref[...] = jnp.zeros_like(o_ref)

    @pl.when(k < nk - 1)
    def _full():
        o_ref[...] += jax.lax.dot_general(
            x_ref[...], w_ref[...], (((1,), (0,)), ((), ())),
            preferred_element_type=jnp.float32,
        )

    @pl.when(k == nk - 1)
    def _tail():
        rem = k_total - (nk - 1) * k_blk
        xb = x_ref[...]
        wb = w_ref[...]
        col = jax.lax.broadcasted_iota(jnp.int32, xb.shape, 1)
        xb = jnp.where(col < rem, xb, 0.0)
        row = jax.lax.broadcasted_iota(jnp.int32, wb.shape, 0)
        wb = jnp.where(row < rem, wb, 0.0)
        o_ref[...] += jax.lax.dot_general(
            xb, wb, (((1,), (0,)), ((), ())),
            preferred_element_type=jnp.float32,
        )


def kernel(x, W):
    m, k_total = x.shape
    _, n = W.shape
    nk = pl.cdiv(k_total, _K_BLK)
    nm = m // _M_BLK
    return pl.pallas_call(
        functools.partial(_mm_body, k_total=k_total, k_blk=_K_BLK, nk=nk),
        grid=(nm, nk),
        in_specs=[
            pl.BlockSpec((_M_BLK, _K_BLK), lambda mi, k: (mi, k)),
            pl.BlockSpec((_K_BLK, n), lambda mi, k: (k, 0)),
        ],
        out_specs=pl.BlockSpec((_M_BLK, n), lambda mi, k: (mi, 0)),
        out_shape=jax.ShapeDtypeStruct((m, n), jnp.float32),
        compiler_params=pltpu.CompilerParams(
            dimension_semantics=("parallel", "arbitrary"),
        ),
    )(x, W)


# speedup vs baseline: 1.2803x; 1.2803x over previous
import functools

import jax
import jax.numpy as jnp
from jax.experimental import pallas as pl
from jax.experimental.pallas import tpu as pltpu

_K_BLK = 1024
_NBUF = 6


def _dot(xb, wb):
    return jax.lax.dot_general(
        xb, wb, (((1,), (0,)), ((), ())), preferred_element_type=jnp.float32
    )


def _body(x_hbm, w_hbm, o_ref, xbufs, wbufs, xtail, wtail, xsems, wsems,
          tsems, *, k_total):
    kb = _K_BLK
    nfull = k_total // kb
    rem = k_total - nfull * kb
    rounds = nfull // _NBUF
    leftover = nfull - rounds * _NBUF

    def x_copy(i, s):
        return pltpu.make_async_copy(
            x_hbm.at[:, pl.ds(i * kb, kb)], xbufs.at[s], xsems.at[s]
        )

    def w_copy(i, s):
        return pltpu.make_async_copy(
            w_hbm.at[pl.ds(i * kb, kb), :], wbufs.at[s], wsems.at[s]
        )

    o_ref[...] = jnp.zeros_like(o_ref)

    for s in range(min(_NBUF, nfull)):
        x_copy(s, s).start()
        w_copy(s, s).start()
    if rem:
        pltpu.make_async_copy(
            x_hbm.at[:, pl.ds(nfull * kb, rem)], xtail, tsems.at[0]
        ).start()
        pltpu.make_async_copy(
            w_hbm.at[pl.ds(nfull * kb, rem), :], wtail, tsems.at[1]
        ).start()

    def process(i, s):
        x_copy(i, s).wait()
        w_copy(i, s).wait()
        o_ref[...] += _dot(xbufs[s], wbufs[s])

    def round_body(r, carry):
        for s in range(_NBUF):
            i = r * _NBUF + s
            process(i, s)
            nxt = i + _NBUF

            @pl.when(nxt < nfull)
            def _refill():
                x_copy(nxt, s).start()
                w_copy(nxt, s).start()
        return carry

    jax.lax.fori_loop(0, rounds, round_body, 0, unroll=False)

    for s in range(leftover):
        process(rounds * _NBUF + s, s)

    if rem:
        pltpu.make_async_copy(
            x_hbm.at[:, pl.ds(nfull * kb, rem)], xtail, tsems.at[0]
        ).wait()
        pltpu.make_async_copy(
            w_hbm.at[pl.ds(nfull * kb, rem), :], wtail, tsems.at[1]
        ).wait()
        o_ref[...] += _dot(xtail[...], wtail[...])


def kernel(x, W):
    m, k_total = x.shape
    _, n = W.shape
    rem = k_total % _K_BLK
    tail_k = rem if rem else _K_BLK
    return pl.pallas_call(
        functools.partial(_body, k_total=k_total),
        in_specs=[
            pl.BlockSpec(memory_space=pltpu.MemorySpace.HBM),
            pl.BlockSpec(memory_space=pltpu.MemorySpace.HBM),
        ],
        out_specs=pl.BlockSpec(memory_space=pltpu.MemorySpace.VMEM),
        out_shape=jax.ShapeDtypeStruct((m, n), jnp.float32),
        scratch_shapes=[
            pltpu.VMEM((_NBUF, m, _K_BLK), jnp.float32),
            pltpu.VMEM((_NBUF, _K_BLK, n), jnp.float32),
            pltpu.VMEM((m, tail_k), jnp.float32),
            pltpu.VMEM((tail_k, n), jnp.float32),
            pltpu.SemaphoreType.DMA((_NBUF,)),
            pltpu.SemaphoreType.DMA((_NBUF,)),
            pltpu.SemaphoreType.DMA((2,)),
        ],
    )(x, W)


# speedup vs baseline: 1.3977x; 1.0917x over previous
import jax
import jax.numpy as jnp
from jax.experimental import pallas as pl
from jax.experimental.pallas import tpu as pltpu

_M_BLK = 32


def _mm_body(x_ref, wt_ref, o_ref):
    o_ref[...] = jax.lax.dot_general(
        x_ref[...], wt_ref[...], (((1,), (1,)), ((), ())),
        preferred_element_type=jnp.float32,
    )


def kernel(x, W):
    m, k_total = x.shape
    _, n = W.shape
    wt = W.T
    nm = m // _M_BLK
    return pl.pallas_call(
        _mm_body,
        grid=(nm,),
        in_specs=[
            pl.BlockSpec((_M_BLK, k_total), lambda mi: (mi, 0)),
            pl.BlockSpec((n, k_total), lambda mi: (0, 0)),
        ],
        out_specs=pl.BlockSpec((_M_BLK, n), lambda mi: (mi, 0)),
        out_shape=jax.ShapeDtypeStruct((m, n), jnp.float32),
        compiler_params=pltpu.CompilerParams(
            dimension_semantics=("arbitrary",),
        ),
    )(x, wt)
